# baseline (speedup 1.0000x reference)
import functools

import jax
import jax.numpy as jnp
from jax import lax
from jax.experimental import pallas as pl
from jax.experimental.pallas import tpu as pltpu
from jax.experimental.pallas import tpu_sc as plsc

T = 2048
D = 768
NH = 12
NKV = 4
HD = 64
E = 16
K = 2
MI = 384
SI = 1024
EPS = 1e-6
REP = NH // NKV

BT = 256
BQ = 256
BM = 128
NP = K * T + E * BM
NBLK = NP // BM


def _dotT(a, b):
    return lax.dot_general(a, b, (((1,), (1,)), ((), ())),
                           preferred_element_type=jnp.float32)


def _rmsnorm(x, w):
    return x * lax.rsqrt(jnp.mean(x * x, axis=1, keepdims=True) + EPS) * w


def _pre_attn_body(h_ref, r_ref, ln_ref, w_ref, b_ref, res_ref, qkv_ref):
    x = h_ref[...] + r_ref[...]
    res_ref[...] = x
    nx = _rmsnorm(x, ln_ref[...])
    qkv_ref[...] = _dotT(nx, w_ref[...]) + b_ref[...]


def _pre_attn(h, r, ln_w, qkv_w, qkv_b):
    nq = (NH + 2 * NKV) * HD
    return pl.pallas_call(
        _pre_attn_body,
        grid=(T // BT,),
        in_specs=[
            pl.BlockSpec((BT, D), lambda i: (i, 0)),
            pl.BlockSpec((BT, D), lambda i: (i, 0)),
            pl.BlockSpec((1, D), lambda i: (0, 0)),
            pl.BlockSpec((nq, D), lambda i: (0, 0)),
            pl.BlockSpec((1, nq), lambda i: (0, 0)),
        ],
        out_specs=[
            pl.BlockSpec((BT, D), lambda i: (i, 0)),
            pl.BlockSpec((BT, nq), lambda i: (i, 0)),
        ],
        out_shape=[
            jax.ShapeDtypeStruct((T, D), jnp.float32),
            jax.ShapeDtypeStruct((T, nq), jnp.float32),
        ],
    )(h, r, ln_w.reshape(1, D), qkv_w, qkv_b.reshape(1, nq))


def _rope(x, cos, sin):
    half = HD // 2
    x1 = x[:, :half]
    x2 = x[:, half:]
    return jnp.concatenate([x1 * cos - x2 * sin, x2 * cos + x1 * sin], axis=1)


def _attn_body(q_ref, k_ref, v_ref, cq_ref, sq_ref, ck_ref, sk_ref, o_ref):
    qr = _rope(q_ref[0], cq_ref[...], sq_ref[...])
    kr = _rope(k_ref[0], ck_ref[...], sk_ref[...])
    s = lax.dot_general(qr, kr, (((1,), (1,)), ((), ())),
                        preferred_element_type=jnp.float32) * (HD ** -0.5)
    row = pl.program_id(1) * BQ + lax.broadcasted_iota(jnp.int32, (BQ, T), 0)
    col = lax.broadcasted_iota(jnp.int32, (BQ, T), 1)
    s = jnp.where(col <= row, s, -1e9)
    m = jnp.max(s, axis=1, keepdims=True)
    p = jnp.exp(s - m)
    p = p / jnp.sum(p, axis=1, keepdims=True)
    o_ref[0] = lax.dot_general(p, v_ref[0], (((1,), (0,)), ((), ())),
                               preferred_element_type=jnp.float32)


def _attention(q3, k3, v3, cos, sin):
    return pl.pallas_call(
        _attn_body,
        grid=(NH, T // BQ),
        in_specs=[
            pl.BlockSpec((1, BQ, HD), lambda h, qb: (h, qb, 0)),
            pl.BlockSpec((1, T, HD), lambda h, qb: (h // REP, 0, 0)),
            pl.BlockSpec((1, T, HD), lambda h, qb: (h // REP, 0, 0)),
            pl.BlockSpec((BQ, HD // 2), lambda h, qb: (qb, 0)),
            pl.BlockSpec((BQ, HD // 2), lambda h, qb: (qb, 0)),
            pl.BlockSpec((T, HD // 2), lambda h, qb: (0, 0)),
            pl.BlockSpec((T, HD // 2), lambda h, qb: (0, 0)),
        ],
        out_specs=pl.BlockSpec((1, BQ, HD), lambda h, qb: (h, qb, 0)),
        out_shape=jax.ShapeDtypeStruct((NH, T, HD), jnp.float32),
    )(q3, k3, v3, cos, sin, cos, sin)


def _post_attn_body(o_ref, ow_ref, r_ref, ln_ref, gw_ref,
                    res_ref, x_ref, probs_ref):
    ao = _dotT(o_ref[...], ow_ref[...])
    x = ao + r_ref[...]
    res_ref[...] = x
    nx = _rmsnorm(x, ln_ref[...])
    x_ref[...] = nx
    logits = _dotT(nx, gw_ref[...])
    m = jnp.max(logits, axis=1, keepdims=True)
    p = jnp.exp(logits - m)
    probs_ref[...] = p / jnp.sum(p, axis=1, keepdims=True)


def _post_attn(o, o_w, res1, ln_w, gate_w):
    return pl.pallas_call(
        _post_attn_body,
        grid=(T // BT,),
        in_specs=[
            pl.BlockSpec((BT, NH * HD), lambda i: (i, 0)),
            pl.BlockSpec((D, NH * HD), lambda i: (0, 0)),
            pl.BlockSpec((BT, D), lambda i: (i, 0)),
            pl.BlockSpec((1, D), lambda i: (0, 0)),
            pl.BlockSpec((E, D), lambda i: (0, 0)),
        ],
        out_specs=[
            pl.BlockSpec((BT, D), lambda i: (i, 0)),
            pl.BlockSpec((BT, D), lambda i: (i, 0)),
            pl.BlockSpec((BT, E), lambda i: (i, 0)),
        ],
        out_shape=[
            jax.ShapeDtypeStruct((T, D), jnp.float32),
            jax.ShapeDtypeStruct((T, D), jnp.float32),
            jax.ShapeDtypeStruct((T, E), jnp.float32),
        ],
    )(o, o_w, res1, ln_w.reshape(1, D), gate_w)


def _sc_gather(table, idx, n_rows, chunk):
    info = plsc.get_sparse_core_info()
    nw = info.num_cores * info.num_subcores
    b_per_w = n_rows // nw
    nchunk = b_per_w // chunk
    mesh = plsc.VectorSubcoreMesh(core_axis_name="c", subcore_axis_name="s")

    @functools.partial(
        pl.kernel, mesh=mesh,
        out_type=jax.ShapeDtypeStruct((n_rows, D), jnp.float32),
        scratch_types=[
            pltpu.VMEM((chunk,), jnp.int32),
            pltpu.VMEM((chunk, D), jnp.float32),
            pltpu.SemaphoreType.DMA,
        ],
    )
    def k(table_hbm, idx_hbm, out_hbm, idx_v, rows_v, sem):
        wid = lax.axis_index("s") * info.num_cores + lax.axis_index("c")
        for c in range(nchunk):
            base = wid * b_per_w + c * chunk
            pltpu.sync_copy(idx_hbm.at[pl.ds(base, chunk)], idx_v)
            pltpu.async_copy(table_hbm.at[idx_v], rows_v, sem).wait()
            pltpu.sync_copy(rows_v, out_hbm.at[pl.ds(base, chunk)])

    return k(table, idx)


def _moe_body(be_ref, x_ref, wgu_ref, wd_ref, y_ref):
    del be_ref
    gu = _dotT(x_ref[...], wgu_ref[0])
    g = gu[:, :MI]
    u = gu[:, MI:]
    hh = g * lax.logistic(g) * u
    y_ref[...] = _dotT(hh, wd_ref[0])


def _moe_matmul(block_e, x_pad, w_gate_up, w_down):
    grid_spec = pltpu.PrefetchScalarGridSpec(
        num_scalar_prefetch=1,
        grid=(NBLK,),
        in_specs=[
            pl.BlockSpec((BM, D), lambda i, be: (i, 0)),
            pl.BlockSpec((1, 2 * MI, D), lambda i, be: (be[i], 0, 0)),
            pl.BlockSpec((1, D, MI), lambda i, be: (be[i], 0, 0)),
        ],
        out_specs=pl.BlockSpec((BM, D), lambda i, be: (i, 0)),
    )
    return pl.pallas_call(
        _moe_body,
        grid_spec=grid_spec,
        out_shape=jax.ShapeDtypeStruct((NP, D), jnp.float32),
    )(block_e, x_pad, w_gate_up, w_down)


def _final_body(x_ref, guw_ref, dw_ref, sgw_ref, c1_ref, c2_ref, w_ref, o_ref):
    x = x_ref[...]
    sgu = _dotT(x, guw_ref[...])
    g = sgu[:, :SI]
    u = sgu[:, SI:]
    sh = g * lax.logistic(g) * u
    shared = _dotT(sh, dw_ref[...])
    sg = lax.logistic(_dotT(x, sgw_ref[...]))
    w = w_ref[...]
    o_ref[...] = (sg * shared
                  + w[:, 0:1] * c1_ref[...]
                  + w[:, 1:2] * c2_ref[...])


def _final(x2, shared_gate_up, shared_down, shared_gate_w, comb, w_pad):
    return pl.pallas_call(
        _final_body,
        grid=(T // BT,),
        in_specs=[
            pl.BlockSpec((BT, D), lambda i: (i, 0)),
            pl.BlockSpec((2 * SI, D), lambda i: (0, 0)),
            pl.BlockSpec((D, SI), lambda i: (0, 0)),
            pl.BlockSpec((1, D), lambda i: (0, 0)),
            pl.BlockSpec((BT, D), lambda i: (i, 0)),
            pl.BlockSpec((BT, D), lambda i: (i + T // BT, 0)),
            pl.BlockSpec((BT, 128), lambda i: (i, 0)),
        ],
        out_specs=pl.BlockSpec((BT, D), lambda i: (i, 0)),
        out_shape=jax.ShapeDtypeStruct((T, D), jnp.float32),
    )(x2, shared_gate_up, shared_down, shared_gate_w, comb, comb, w_pad)


def kernel(positions, hidden_states, residual, input_ln_w, qkv_w, qkv_b, o_w,
           post_ln_w, gate_w, w_gate_up, w_down, shared_gate_up, shared_down,
           shared_gate_w):
    half = HD // 2
    inv = 1.0 / (10000.0 ** (jnp.arange(half, dtype=jnp.float32) / half))
    fr = positions.astype(jnp.float32)[:, None] * inv[None, :]
    cos = jnp.cos(fr)
    sin = jnp.sin(fr)

    res1, qkv = _pre_attn(hidden_states, residual, input_ln_w, qkv_w, qkv_b)
    q3 = qkv[:, :NH * HD].reshape(T, NH, HD).transpose(1, 0, 2)
    k3 = qkv[:, NH * HD:(NH + NKV) * HD].reshape(T, NKV, HD).transpose(1, 0, 2)
    v3 = qkv[:, (NH + NKV) * HD:].reshape(T, NKV, HD).transpose(1, 0, 2)
    attn_o3 = _attention(q3, k3, v3, cos, sin)
    attn_o = attn_o3.transpose(1, 0, 2).reshape(T, NH * HD)
    res2, x2, probs = _post_attn(attn_o, o_w, res1, post_ln_w, gate_w)

    topv, topi = lax.top_k(probs, K)
    topv = topv / jnp.sum(topv, axis=-1, keepdims=True)
    flat_e = topi.reshape(-1).astype(jnp.int32)
    onehot = (flat_e[:, None] == jnp.arange(E, dtype=jnp.int32)[None, :])
    oh = onehot.astype(jnp.int32)
    counts = jnp.sum(oh, axis=0)
    rank = jnp.cumsum(oh, axis=0) - 1
    rank_p = jnp.take_along_axis(rank, flat_e[:, None], axis=1)[:, 0]
    nblk = (counts + BM - 1) // BM
    pstart = (jnp.concatenate([jnp.zeros((1,), jnp.int32),
                               jnp.cumsum(nblk)[:-1].astype(jnp.int32)])
              * BM)
    pos = pstart[flat_e] + rank_p
    tok = jnp.arange(T * K, dtype=jnp.int32) // K
    src_tok = jnp.zeros((NP,), jnp.int32).at[pos].set(tok)
    boff = jnp.arange(NBLK, dtype=jnp.int32) * BM
    block_e = (jnp.sum(boff[:, None] >= pstart[None, :], axis=1) - 1
               ).astype(jnp.int32)

    x_pad = _sc_gather(x2, src_tok, NP, chunk=96)
    y_pad = _moe_matmul(block_e, x_pad, w_gate_up, w_down)

    pos_pair = pos.reshape(T, K)
    idx_all = jnp.concatenate([pos_pair[:, 0], pos_pair[:, 1]])
    comb = _sc_gather(y_pad, idx_all, K * T, chunk=128)

    w_pad = jnp.pad(topv, ((0, 0), (0, 128 - K)))
    out = _final(x2, shared_gate_up, shared_down,
                 shared_gate_w.reshape(1, D), comb, w_pad)
    return out, res2


# speedup vs baseline: 1.2625x; 1.2625x over previous
import functools

import jax
import jax.numpy as jnp
from jax import lax
from jax.experimental import pallas as pl
from jax.experimental.pallas import tpu as pltpu
from jax.experimental.pallas import tpu_sc as plsc

T = 2048
D = 768
NH = 12
NKV = 4
HD = 64
E = 16
K = 2
MI = 384
SI = 1024
EPS = 1e-6
REP = NH // NKV

BT = 256
BQ = 256
BM = 128
NP = K * T + E * BM
NBLK = NP // BM


def _dotT(a, b):
    return lax.dot_general(a, b, (((1,), (1,)), ((), ())),
                           preferred_element_type=jnp.float32)


def _rmsnorm(x, w):
    return x * lax.rsqrt(jnp.mean(x * x, axis=1, keepdims=True) + EPS) * w


def _pre_attn_body(h_ref, r_ref, ln_ref, w_ref, b_ref, res_ref, qkv_ref):
    x = h_ref[...] + r_ref[...]
    res_ref[...] = x
    nx = _rmsnorm(x, ln_ref[...])
    qkv_ref[...] = _dotT(nx, w_ref[...]) + b_ref[...]


def _pre_attn(h, r, ln_w, qkv_w, qkv_b):
    nq = (NH + 2 * NKV) * HD
    return pl.pallas_call(
        _pre_attn_body,
        grid=(T // BT,),
        in_specs=[
            pl.BlockSpec((BT, D), lambda i: (i, 0)),
            pl.BlockSpec((BT, D), lambda i: (i, 0)),
            pl.BlockSpec((1, D), lambda i: (0, 0)),
            pl.BlockSpec((nq, D), lambda i: (0, 0)),
            pl.BlockSpec((1, nq), lambda i: (0, 0)),
        ],
        out_specs=[
            pl.BlockSpec((BT, D), lambda i: (i, 0)),
            pl.BlockSpec((BT, nq), lambda i: (i, 0)),
        ],
        out_shape=[
            jax.ShapeDtypeStruct((T, D), jnp.float32),
            jax.ShapeDtypeStruct((T, nq), jnp.float32),
        ],
    )(h, r, ln_w.reshape(1, D), qkv_w, qkv_b.reshape(1, nq))


def _rope(x, cos, sin):
    half = HD // 2
    x1 = x[:, :half]
    x2 = x[:, half:]
    return jnp.concatenate([x1 * cos - x2 * sin, x2 * cos + x1 * sin], axis=1)


def _attn_body(q_ref, k_ref, v_ref, cq_ref, sq_ref, ck_ref, sk_ref, o_ref):
    qr = _rope(q_ref[0], cq_ref[...], sq_ref[...])
    kr = _rope(k_ref[0], ck_ref[...], sk_ref[...])
    s = lax.dot_general(qr, kr, (((1,), (1,)), ((), ())),
                        preferred_element_type=jnp.float32) * (HD ** -0.5)
    row = pl.program_id(1) * BQ + lax.broadcasted_iota(jnp.int32, (BQ, T), 0)
    col = lax.broadcasted_iota(jnp.int32, (BQ, T), 1)
    s = jnp.where(col <= row, s, -1e9)
    m = jnp.max(s, axis=1, keepdims=True)
    p = jnp.exp(s - m)
    p = p / jnp.sum(p, axis=1, keepdims=True)
    o_ref[0] = lax.dot_general(p, v_ref[0], (((1,), (0,)), ((), ())),
                               preferred_element_type=jnp.float32)


def _attention(q3, k3, v3, cos, sin):
    return pl.pallas_call(
        _attn_body,
        grid=(NH, T // BQ),
        in_specs=[
            pl.BlockSpec((1, BQ, HD), lambda h, qb: (h, qb, 0)),
            pl.BlockSpec((1, T, HD), lambda h, qb: (h // REP, 0, 0)),
            pl.BlockSpec((1, T, HD), lambda h, qb: (h // REP, 0, 0)),
            pl.BlockSpec((BQ, HD // 2), lambda h, qb: (qb, 0)),
            pl.BlockSpec((BQ, HD // 2), lambda h, qb: (qb, 0)),
            pl.BlockSpec((T, HD // 2), lambda h, qb: (0, 0)),
            pl.BlockSpec((T, HD // 2), lambda h, qb: (0, 0)),
        ],
        out_specs=pl.BlockSpec((1, BQ, HD), lambda h, qb: (h, qb, 0)),
        out_shape=jax.ShapeDtypeStruct((NH, T, HD), jnp.float32),
    )(q3, k3, v3, cos, sin, cos, sin)


def _post_attn_body(o_ref, ow_ref, r_ref, ln_ref, gw_ref,
                    res_ref, x_ref, wd_ref):
    ao = _dotT(o_ref[...], ow_ref[...])
    x = ao + r_ref[...]
    res_ref[...] = x
    nx = _rmsnorm(x, ln_ref[...])
    x_ref[...] = nx
    logits = _dotT(nx, gw_ref[...])
    m = jnp.max(logits, axis=1, keepdims=True)
    p = jnp.exp(logits - m)
    p = p / jnp.sum(p, axis=1, keepdims=True)
    iot = lax.broadcasted_iota(jnp.int32, (BT, E), 1)
    m1 = jnp.max(p, axis=1, keepdims=True)
    i1 = jnp.min(jnp.where(p == m1, iot, E), axis=1, keepdims=True)
    p2 = jnp.where(iot == i1, -1.0, p)
    m2 = jnp.max(p2, axis=1, keepdims=True)
    i2 = jnp.min(jnp.where(p2 == m2, iot, E), axis=1, keepdims=True)
    s = m1 + m2
    wd_ref[...] = (jnp.where(iot == i1, m1 / s, 0.0)
                   + jnp.where(iot == i2, m2 / s, 0.0))


def _post_attn(o, o_w, res1, ln_w, gate_w):
    return pl.pallas_call(
        _post_attn_body,
        grid=(T // BT,),
        in_specs=[
            pl.BlockSpec((BT, NH * HD), lambda i: (i, 0)),
            pl.BlockSpec((D, NH * HD), lambda i: (0, 0)),
            pl.BlockSpec((BT, D), lambda i: (i, 0)),
            pl.BlockSpec((1, D), lambda i: (0, 0)),
            pl.BlockSpec((E, D), lambda i: (0, 0)),
        ],
        out_specs=[
            pl.BlockSpec((BT, D), lambda i: (i, 0)),
            pl.BlockSpec((BT, D), lambda i: (i, 0)),
            pl.BlockSpec((BT, E), lambda i: (i, 0)),
        ],
        out_shape=[
            jax.ShapeDtypeStruct((T, D), jnp.float32),
            jax.ShapeDtypeStruct((T, D), jnp.float32),
            jax.ShapeDtypeStruct((T, E), jnp.float32),
        ],
    )(o, o_w, res1, ln_w.reshape(1, D), gate_w)


def _sc_gather(table, idx, n_rows, chunk):
    info = plsc.get_sparse_core_info()
    nw = info.num_cores * info.num_subcores
    b_per_w = n_rows // nw
    nchunk = b_per_w // chunk
    mesh = plsc.VectorSubcoreMesh(core_axis_name="c", subcore_axis_name="s")

    @functools.partial(
        pl.kernel, mesh=mesh,
        out_type=jax.ShapeDtypeStruct((n_rows, D), jnp.float32),
        scratch_types=[
            pltpu.VMEM((chunk,), jnp.int32),
            pltpu.VMEM((chunk, D), jnp.float32),
            pltpu.SemaphoreType.DMA,
        ],
    )
    def k(table_hbm, idx_hbm, out_hbm, idx_v, rows_v, sem):
        wid = lax.axis_index("s") * info.num_cores + lax.axis_index("c")
        for c in range(nchunk):
            base = wid * b_per_w + c * chunk
            pltpu.sync_copy(idx_hbm.at[pl.ds(base, chunk)], idx_v)
            pltpu.async_copy(table_hbm.at[idx_v], rows_v, sem).wait()
            pltpu.sync_copy(rows_v, out_hbm.at[pl.ds(base, chunk)])

    return k(table, idx)


def _sc_dispatch(x2, tok, pos):
    info = plsc.get_sparse_core_info()
    nw = info.num_cores * info.num_subcores
    b_per_w = (K * T) // nw
    mesh = plsc.VectorSubcoreMesh(core_axis_name="c", subcore_axis_name="s")

    @functools.partial(
        pl.kernel, mesh=mesh,
        out_type=jax.ShapeDtypeStruct((NP, D), jnp.float32),
        scratch_types=[
            pltpu.VMEM((b_per_w,), jnp.int32),
            pltpu.VMEM((b_per_w,), jnp.int32),
            pltpu.VMEM((b_per_w, D), jnp.float32),
            pltpu.SemaphoreType.DMA,
            pltpu.SemaphoreType.DMA,
        ],
    )
    def k(x2_hbm, tok_hbm, pos_hbm, out_hbm, tok_v, pos_v, rows_v, s1, s2):
        wid = lax.axis_index("s") * info.num_cores + lax.axis_index("c")
        base = wid * b_per_w
        pltpu.sync_copy(tok_hbm.at[pl.ds(base, b_per_w)], tok_v)
        pltpu.sync_copy(pos_hbm.at[pl.ds(base, b_per_w)], pos_v)
        pltpu.async_copy(x2_hbm.at[tok_v], rows_v, s1).wait()
        pltpu.async_copy(rows_v, out_hbm.at[pos_v], s2).wait()

    return k(x2, tok, pos)


def _moe_body(be_ref, x_ref, wgu_ref, wd_ref, y_ref):
    del be_ref
    gu = _dotT(x_ref[...], wgu_ref[0])
    g = gu[:, :MI]
    u = gu[:, MI:]
    hh = g * lax.logistic(g) * u
    y_ref[...] = _dotT(hh, wd_ref[0])


def _moe_matmul(block_e, x_pad, w_gate_up, w_down):
    grid_spec = pltpu.PrefetchScalarGridSpec(
        num_scalar_prefetch=1,
        grid=(NBLK,),
        in_specs=[
            pl.BlockSpec((BM, D), lambda i, be: (i, 0)),
            pl.BlockSpec((1, 2 * MI, D), lambda i, be: (be[i], 0, 0)),
            pl.BlockSpec((1, D, MI), lambda i, be: (be[i], 0, 0)),
        ],
        out_specs=pl.BlockSpec((BM, D), lambda i, be: (i, 0)),
    )
    return pl.pallas_call(
        _moe_body,
        grid_spec=grid_spec,
        out_shape=jax.ShapeDtypeStruct((NP, D), jnp.float32),
    )(block_e, x_pad, w_gate_up, w_down)


def _final_body(x_ref, guw_ref, dw_ref, sgw_ref, c1_ref, c2_ref, w_ref, o_ref):
    x = x_ref[...]
    sgu = _dotT(x, guw_ref[...])
    g = sgu[:, :SI]
    u = sgu[:, SI:]
    sh = g * lax.logistic(g) * u
    shared = _dotT(sh, dw_ref[...])
    sg = lax.logistic(_dotT(x, sgw_ref[...]))
    w = w_ref[...]
    o_ref[...] = (sg * shared
                  + w[:, 0:1] * c1_ref[...]
                  + w[:, 1:2] * c2_ref[...])


def _final(x2, shared_gate_up, shared_down, shared_gate_w, comb, w_pad):
    return pl.pallas_call(
        _final_body,
        grid=(T // BT,),
        in_specs=[
            pl.BlockSpec((BT, D), lambda i: (i, 0)),
            pl.BlockSpec((2 * SI, D), lambda i: (0, 0)),
            pl.BlockSpec((D, SI), lambda i: (0, 0)),
            pl.BlockSpec((1, D), lambda i: (0, 0)),
            pl.BlockSpec((BT, D), lambda i: (i, 0)),
            pl.BlockSpec((BT, D), lambda i: (i + T // BT, 0)),
            pl.BlockSpec((BT, 128), lambda i: (i, 0)),
        ],
        out_specs=pl.BlockSpec((BT, D), lambda i: (i, 0)),
        out_shape=jax.ShapeDtypeStruct((T, D), jnp.float32),
    )(x2, shared_gate_up, shared_down, shared_gate_w, comb, comb, w_pad)


def kernel(positions, hidden_states, residual, input_ln_w, qkv_w, qkv_b, o_w,
           post_ln_w, gate_w, w_gate_up, w_down, shared_gate_up, shared_down,
           shared_gate_w):
    half = HD // 2
    inv = 1.0 / (10000.0 ** (jnp.arange(half, dtype=jnp.float32) / half))
    fr = positions.astype(jnp.float32)[:, None] * inv[None, :]
    cos = jnp.cos(fr)
    sin = jnp.sin(fr)

    res1, qkv = _pre_attn(hidden_states, residual, input_ln_w, qkv_w, qkv_b)
    q3 = qkv[:, :NH * HD].reshape(T, NH, HD).transpose(1, 0, 2)
    k3 = qkv[:, NH * HD:(NH + NKV) * HD].reshape(T, NKV, HD).transpose(1, 0, 2)
    v3 = qkv[:, (NH + NKV) * HD:].reshape(T, NKV, HD).transpose(1, 0, 2)
    attn_o3 = _attention(q3, k3, v3, cos, sin)
    attn_o = attn_o3.transpose(1, 0, 2).reshape(T, NH * HD)
    res2, x2, wd = _post_attn(attn_o, o_w, res1, post_ln_w, gate_w)

    iota_e = jnp.arange(E, dtype=jnp.int32)[None, :]
    chosen = wd > 0.0
    e1 = jnp.min(jnp.where(chosen, iota_e, E), axis=1)
    e2 = jnp.max(jnp.where(chosen, iota_e, -1), axis=1)
    w1 = jnp.sum(jnp.where(iota_e == e1[:, None], wd, 0.0), axis=1)
    w2 = jnp.sum(jnp.where(iota_e == e2[:, None], wd, 0.0), axis=1)
    flat_e = jnp.stack([e1, e2], axis=1).reshape(-1)
    oh = (flat_e[:, None] == iota_e).astype(jnp.int32)
    counts = jnp.sum(oh, axis=0)
    rank = jnp.cumsum(oh, axis=0) - 1
    rank_p = jnp.sum(rank * oh, axis=1)
    nblk = (counts + BM - 1) // BM
    pstart = (jnp.concatenate([jnp.zeros((1,), jnp.int32),
                               jnp.cumsum(nblk)[:-1].astype(jnp.int32)])
              * BM)
    pstart_p = jnp.sum(pstart[None, :] * oh, axis=1)
    pos = pstart_p + rank_p
    tok = jnp.arange(T * K, dtype=jnp.int32) // K
    boff = jnp.arange(NBLK, dtype=jnp.int32) * BM
    block_e = (jnp.sum((boff[:, None] >= pstart[None, :]).astype(jnp.int32),
                       axis=1) - 1).astype(jnp.int32)

    x_pad = _sc_dispatch(x2, tok, pos)
    y_pad = _moe_matmul(block_e, x_pad, w_gate_up, w_down)

    pos_pair = pos.reshape(T, K)
    idx_all = jnp.concatenate([pos_pair[:, 0], pos_pair[:, 1]])
    comb = _sc_gather(y_pad, idx_all, K * T, chunk=128)

    w_pad = jnp.pad(jnp.stack([w1, w2], axis=1), ((0, 0), (0, 128 - K)))
    out = _final(x2, shared_gate_up, shared_down,
                 shared_gate_w.reshape(1, D), comb, w_pad)
    return out, res2
